# baseline (speedup 1.0000x reference)
import functools

import jax
import jax.numpy as jnp
from jax import lax
from jax.experimental import pallas as pl
from jax.experimental.pallas import tpu as pltpu
from jax.experimental.pallas import tpu_sc as plsc

_NW = 32


def _sc_gather_a_body(k, ew, h_hbm, src_hbm, a_hbm, src_v, abuf0, abuf1,
                      sem_a0, sem_a1):
    wid = lax.axis_index("s") * 2 + lax.axis_index("c")
    bufs = (abuf0, abuf1)
    sems = (sem_a0, sem_a1)

    def chunk(i, carry):
        base = wid * ew + i * 2 * k
        for b in range(2):
            pltpu.sync_copy(src_hbm.at[pl.ds(base + b * k, k)], src_v)
            cp = pltpu.async_copy(h_hbm.at[src_v], bufs[b], sems[b])
            cp.wait()
            pltpu.sync_copy(bufs[b], a_hbm.at[pl.ds(base + b * k, k)])
        return carry

    lax.fori_loop(0, ew // (2 * k), chunk, 0)


def _sc_gather_a(h, src, k=256):
    e = src.shape[0]
    fx = h.shape[1]
    ep = (e + _NW * 2 * k - 1) // (_NW * 2 * k) * (_NW * 2 * k)
    if ep != e:
        src = jnp.pad(src, (0, ep - e))
    ew = ep // _NW
    mesh = plsc.VectorSubcoreMesh(core_axis_name="c", subcore_axis_name="s")
    fn = pl.kernel(
        functools.partial(_sc_gather_a_body, k, ew),
        out_type=jax.ShapeDtypeStruct((ep, fx), jnp.float32),
        mesh=mesh,
        scratch_types=[
            pltpu.VMEM((k,), jnp.int32),
            pltpu.VMEM((k, fx), jnp.float32),
            pltpu.VMEM((k, fx), jnp.float32),
            pltpu.SemaphoreType.DMA,
            pltpu.SemaphoreType.DMA,
        ],
    )
    return fn(h, src)


def _mlp3_ad_body(a_ref, d_ref, w0a_ref, w0d_ref, b0_ref, w1_ref, b1_ref,
                  w2_ref, b2_ref, out_ref):
    h = jnp.dot(a_ref[...], w0a_ref[...], preferred_element_type=jnp.float32)
    h = h + jnp.dot(d_ref[...], w0d_ref[...], preferred_element_type=jnp.float32)
    h = jnp.maximum(h + b0_ref[...], 0.0)
    h = jnp.dot(h, w1_ref[...], preferred_element_type=jnp.float32)
    h = jnp.maximum(h + b1_ref[...], 0.0)
    h = jnp.dot(h, w2_ref[...], preferred_element_type=jnp.float32)
    out_ref[...] = h + b2_ref[...]


def _mlp3_d_body(d_ref, w0d_ref, b0_ref, w1_ref, b1_ref, w2_ref, b2_ref,
                 out_ref):
    h = jnp.dot(d_ref[...], w0d_ref[...], preferred_element_type=jnp.float32)
    h = jnp.maximum(h + b0_ref[...], 0.0)
    h = jnp.dot(h, w1_ref[...], preferred_element_type=jnp.float32)
    h = jnp.maximum(h + b1_ref[...], 0.0)
    h = jnp.dot(h, w2_ref[...], preferred_element_type=jnp.float32)
    out_ref[...] = h + b2_ref[...]


def _mlp3(a, d, w0a, w0d, b0, w1, b1, w2, b2, chunk=2048):
    n = d.shape[0]
    f1 = w1.shape[0]
    f2 = w2.shape[0]
    f3 = w2.shape[1]
    n_pad = (n + chunk - 1) // chunk * chunk
    if n_pad != n:
        d = jnp.pad(d, ((0, n_pad - n), (0, 0)))
        if a is not None:
            a = jnp.pad(a, ((0, n_pad - n), (0, 0)))
    grid = n_pad // chunk
    wspecs = [
        pl.BlockSpec((16, f1), lambda i: (0, 0)),
        pl.BlockSpec((1, f1), lambda i: (0, 0)),
        pl.BlockSpec((f1, f2), lambda i: (0, 0)),
        pl.BlockSpec((1, f2), lambda i: (0, 0)),
        pl.BlockSpec((f2, f3), lambda i: (0, 0)),
        pl.BlockSpec((1, f3), lambda i: (0, 0)),
    ]
    wargs = [w0d, b0.reshape(1, -1), w1, b1.reshape(1, -1), w2,
             b2.reshape(1, -1)]
    if a is None:
        body = _mlp3_d_body
        in_specs = [pl.BlockSpec((chunk, 16), lambda i: (i, 0))] + wspecs
        args = [d] + wargs
    else:
        body = _mlp3_ad_body
        fx = a.shape[1]
        in_specs = ([pl.BlockSpec((chunk, fx), lambda i: (i, 0)),
                     pl.BlockSpec((chunk, 16), lambda i: (i, 0)),
                     pl.BlockSpec((fx, f1), lambda i: (0, 0))]
                    + wspecs)
        args = [a, d, w0a] + wargs
    out = pl.pallas_call(
        body,
        grid=(grid,),
        in_specs=in_specs,
        out_specs=pl.BlockSpec((chunk, f3), lambda i: (i, 0)),
        out_shape=jax.ShapeDtypeStruct((n_pad, f3), jnp.float32),
    )(*args)
    return out[:n]


def _point_conv(x, pos, edge_index, p, pre, add_self_loops, num_nodes,
                n_chunks, k):
    src = edge_index[0]
    dst = edge_index[1]
    if add_self_loops:
        loop = jnp.arange(num_nodes, dtype=src.dtype)
        src = jnp.concatenate([src, loop])
        dst = jnp.concatenate([dst, loop])
    E = src.shape[0]
    fx = x.shape[1]
    w0 = p[pre + '_W0']
    if add_self_loops:
        w0a = None
        w0d = jnp.pad(w0, ((0, 16 - w0.shape[0]), (0, 0)))
    else:
        w0a = w0[:fx]
        w0d = jnp.pad(w0[fx:], ((3, 10), (0, 0)))
    b0 = p[pre + '_b0']
    w1, b1 = p[pre + '_W1'], p[pre + '_b1']
    w2, b2 = p[pre + '_W2'], p[pre + '_b2']

    bounds = [min(((E * i // n_chunks) + 8191) // 8192 * 8192, E)
              for i in range(n_chunks + 1)]
    bounds[0] = 0
    bounds[-1] = E
    agg = None
    for i in range(n_chunks):
        lo, hi = bounds[i], bounds[i + 1]
        if hi <= lo:
            continue
        s, dvec = src[lo:hi], dst[lo:hi]
        diff = pos[s] - pos[dvec]
        if add_self_loops:
            d_e = jnp.pad(jnp.concatenate([x[s], diff], axis=1),
                          ((0, 0), (0, 10)))
            a_e = None
        else:
            d_e = jnp.pad(diff, ((0, 0), (3, 10)))
            a_e = x[s]
        m = _mlp3(a_e, d_e, w0a, w0d, b0, w1, b1, w2, b2)[:hi - lo]
        part = jax.ops.segment_max(m, dvec, num_segments=num_nodes)
        agg = part if agg is None else jnp.maximum(agg, part)
    return jnp.where(jnp.isfinite(agg), agg, 0.0)


def _bn(h, g, b):
    m = h.mean(0)
    v = h.var(0)
    return (h - m) / jnp.sqrt(v + 1e-05) * g + b


def kernel(x, pos, params, edge_index, batch, pool_perm1, edge_index2, pool_perm2, edge_index3):
    N = x.shape[0]
    h = _point_conv(x, pos, edge_index, params, 'b1', True, N,
                    n_chunks=4, k=512)
    h = h[pool_perm1]
    pos2 = pos[pool_perm1]
    batch2 = batch[pool_perm1]
    h = _point_conv(h, pos2, edge_index2, params, 'b2', False,
                    pool_perm1.shape[0], n_chunks=4, k=256)
    h = h[pool_perm2]
    pos3 = pos2[pool_perm2]
    batch3 = batch2[pool_perm2]
    h = _point_conv(h, pos3, edge_index3, params, 'b3', False,
                    pool_perm2.shape[0], n_chunks=4, k=128)
    g = jax.ops.segment_max(h, batch3, num_segments=16)
    g = jnp.where(jnp.isfinite(g), g, 0.0)
    out = jax.nn.relu(_bn(g, params['bn1_g'], params['bn1_b']))
    out = out @ params['m_W1'] + params['m_b1']
    out = jax.nn.relu(_bn(out, params['bn2_g'], params['bn2_b']))
    out = out @ params['m_W2'] + params['m_b2']
    out = jax.nn.relu(_bn(out, params['bn3_g'], params['bn3_b']))
    out = out @ params['m_W3'] + params['m_b3']
    return out


# speedup vs baseline: 1.1082x; 1.1082x over previous
import functools

import jax
import jax.numpy as jnp
from jax import lax
from jax.experimental import pallas as pl
from jax.experimental.pallas import tpu as pltpu
from jax.experimental.pallas import tpu_sc as plsc

_NW = 32


def _sc_gather_a_body(k, ew, h_hbm, src_hbm, a_hbm, src_v, abuf0, abuf1,
                      sem_a0, sem_a1):
    wid = lax.axis_index("s") * 2 + lax.axis_index("c")
    bufs = (abuf0, abuf1)
    sems = (sem_a0, sem_a1)

    def chunk(i, carry):
        base = wid * ew + i * 2 * k
        for b in range(2):
            pltpu.sync_copy(src_hbm.at[pl.ds(base + b * k, k)], src_v)
            cp = pltpu.async_copy(h_hbm.at[src_v], bufs[b], sems[b])
            cp.wait()
            pltpu.sync_copy(bufs[b], a_hbm.at[pl.ds(base + b * k, k)])
        return carry

    lax.fori_loop(0, ew // (2 * k), chunk, 0)


def _sc_gather_a(h, src, k=256):
    e = src.shape[0]
    fx = h.shape[1]
    ep = (e + _NW * 2 * k - 1) // (_NW * 2 * k) * (_NW * 2 * k)
    if ep != e:
        src = jnp.pad(src, (0, ep - e))
    ew = ep // _NW
    mesh = plsc.VectorSubcoreMesh(core_axis_name="c", subcore_axis_name="s")
    fn = pl.kernel(
        functools.partial(_sc_gather_a_body, k, ew),
        out_type=jax.ShapeDtypeStruct((ep, fx), jnp.float32),
        mesh=mesh,
        scratch_types=[
            pltpu.VMEM((k,), jnp.int32),
            pltpu.VMEM((k, fx), jnp.float32),
            pltpu.VMEM((k, fx), jnp.float32),
            pltpu.SemaphoreType.DMA,
            pltpu.SemaphoreType.DMA,
        ],
    )
    return fn(h, src)


def _mlp3_ad_body(a_ref, d_ref, w0a_ref, w0d_ref, b0_ref, w1_ref, b1_ref,
                  w2_ref, b2_ref, out_ref):
    h = jnp.dot(a_ref[...], w0a_ref[...], preferred_element_type=jnp.float32)
    h = h + jnp.dot(d_ref[...], w0d_ref[...], preferred_element_type=jnp.float32)
    h = jnp.maximum(h + b0_ref[...], 0.0)
    h = jnp.dot(h, w1_ref[...], preferred_element_type=jnp.float32)
    h = jnp.maximum(h + b1_ref[...], 0.0)
    h = jnp.dot(h, w2_ref[...], preferred_element_type=jnp.float32)
    out_ref[...] = h + b2_ref[...]


def _mlp3_d_body(d_ref, w0d_ref, b0_ref, w1_ref, b1_ref, w2_ref, b2_ref,
                 out_ref):
    h = jnp.dot(d_ref[...], w0d_ref[...], preferred_element_type=jnp.float32)
    h = jnp.maximum(h + b0_ref[...], 0.0)
    h = jnp.dot(h, w1_ref[...], preferred_element_type=jnp.float32)
    h = jnp.maximum(h + b1_ref[...], 0.0)
    h = jnp.dot(h, w2_ref[...], preferred_element_type=jnp.float32)
    out_ref[...] = h + b2_ref[...]


def _mlp3(a, d, w0a, w0d, b0, w1, b1, w2, b2, chunk=2048):
    n = d.shape[0]
    f1 = w1.shape[0]
    f2 = w2.shape[0]
    f3 = w2.shape[1]
    assert n % chunk == 0
    n_pad = n
    grid = n_pad // chunk
    wspecs = [
        pl.BlockSpec((16, f1), lambda i: (0, 0)),
        pl.BlockSpec((1, f1), lambda i: (0, 0)),
        pl.BlockSpec((f1, f2), lambda i: (0, 0)),
        pl.BlockSpec((1, f2), lambda i: (0, 0)),
        pl.BlockSpec((f2, f3), lambda i: (0, 0)),
        pl.BlockSpec((1, f3), lambda i: (0, 0)),
    ]
    wargs = [w0d, b0.reshape(1, -1), w1, b1.reshape(1, -1), w2,
             b2.reshape(1, -1)]
    if a is None:
        body = _mlp3_d_body
        in_specs = [pl.BlockSpec((chunk, 16), lambda i: (i, 0))] + wspecs
        args = [d] + wargs
    else:
        body = _mlp3_ad_body
        fx = a.shape[1]
        in_specs = ([pl.BlockSpec((chunk, fx), lambda i: (i, 0)),
                     pl.BlockSpec((chunk, 16), lambda i: (i, 0)),
                     pl.BlockSpec((fx, f1), lambda i: (0, 0))]
                    + wspecs)
        args = [a, d, w0a] + wargs
    out = pl.pallas_call(
        body,
        grid=(grid,),
        in_specs=in_specs,
        out_specs=pl.BlockSpec((chunk, f3), lambda i: (i, 0)),
        out_shape=jax.ShapeDtypeStruct((n_pad, f3), jnp.float32),
    )(*args)
    return out


def _point_conv(x, pos, edge_index, p, pre, add_self_loops, num_nodes,
                n_chunks, k):
    src = edge_index[0]
    dst = edge_index[1]
    if add_self_loops:
        loop = jnp.arange(num_nodes, dtype=src.dtype)
        src = jnp.concatenate([src, loop])
        dst = jnp.concatenate([dst, loop])
    E = src.shape[0]
    fx = x.shape[1]
    w0 = p[pre + '_W0']
    if add_self_loops:
        w0a = None
        w0d = jnp.pad(w0, ((0, 16 - w0.shape[0]), (0, 0)))
        t1 = jnp.pad(jnp.concatenate([x, pos], axis=1), ((0, 0), (0, 10)))
    else:
        w0a = w0[:fx]
        w0d = jnp.pad(w0[fx:], ((3, 10), (0, 0)))
        t1 = jnp.pad(pos, ((0, 0), (3, 10)))
    t2n = jnp.pad(-pos, ((0, 0), (3, 10)))
    b0 = p[pre + '_b0']
    w1, b1 = p[pre + '_W1'], p[pre + '_b1']
    w2, b2 = p[pre + '_W2'], p[pre + '_b2']

    align = 8192
    ep = (E + n_chunks * align - 1) // (n_chunks * align) * (n_chunks * align)
    src = jnp.pad(src, (0, ep - E))
    dst = jnp.pad(dst, (0, ep - E), constant_values=num_nodes)
    cs = ep // n_chunks
    agg = None
    for i in range(n_chunks):
        s = lax.dynamic_slice_in_dim(src, i * cs, cs)
        dvec = lax.dynamic_slice_in_dim(dst, i * cs, cs)
        d_e = t1[s] + t2n[dvec]
        a_e = None if add_self_loops else x[s]
        m = _mlp3(a_e, d_e, w0a, w0d, b0, w1, b1, w2, b2)
        part = jax.ops.segment_max(m, dvec, num_segments=num_nodes)
        agg = part if agg is None else jnp.maximum(agg, part)
    return jnp.where(jnp.isfinite(agg), agg, 0.0)


def _bn(h, g, b):
    m = h.mean(0)
    v = h.var(0)
    return (h - m) / jnp.sqrt(v + 1e-05) * g + b


def kernel(x, pos, params, edge_index, batch, pool_perm1, edge_index2, pool_perm2, edge_index3):
    N = x.shape[0]
    h = _point_conv(x, pos, edge_index, params, 'b1', True, N,
                    n_chunks=4, k=512)
    h = h[pool_perm1]
    pos2 = pos[pool_perm1]
    batch2 = batch[pool_perm1]
    h = _point_conv(h, pos2, edge_index2, params, 'b2', False,
                    pool_perm1.shape[0], n_chunks=4, k=256)
    h = h[pool_perm2]
    pos3 = pos2[pool_perm2]
    batch3 = batch2[pool_perm2]
    h = _point_conv(h, pos3, edge_index3, params, 'b3', False,
                    pool_perm2.shape[0], n_chunks=4, k=128)
    g = jax.ops.segment_max(h, batch3, num_segments=16)
    g = jnp.where(jnp.isfinite(g), g, 0.0)
    out = jax.nn.relu(_bn(g, params['bn1_g'], params['bn1_b']))
    out = out @ params['m_W1'] + params['m_b1']
    out = jax.nn.relu(_bn(out, params['bn2_g'], params['bn2_b']))
    out = out @ params['m_W2'] + params['m_b2']
    out = jax.nn.relu(_bn(out, params['bn3_g'], params['bn3_b']))
    out = out @ params['m_W3'] + params['m_b3']
    return out
